# baseline (speedup 1.0000x reference)
import functools

import jax
import jax.numpy as jnp
from jax import lax
from jax.experimental import pallas as pl
from jax.experimental.pallas import tpu as pltpu
from jax.experimental.pallas import tpu_sc as plsc

NC = 2
NS = 16
NW = NC * NS
B = 16384
D = 16
BPW = B // NW
NG = BPW // 16


@functools.cache
def _mesh():
    return plsc.VectorSubcoreMesh(
        core_axis_name="c", subcore_axis_name="s", num_cores=NC, num_subcores=NS
    )


def _sc_body(idx_hbm, nodeT, relaT, linkT, err_hbm, *scratch):
    idx_v = scratch[0]
    tbuf = scratch[1]
    out_v = scratch[2]
    sems = scratch[3:5]
    slabs = scratch[5:]
    wid = lax.axis_index("s") * NC + lax.axis_index("c")
    pltpu.sync_copy(idx_hbm.at[wid], idx_v)

    tabs = (relaT, linkT, nodeT, nodeT, nodeT, nodeT)
    cols = (0, 0, 1, 2, 3, 4)
    iota = lax.iota(jnp.int32, 16)

    def issue_wave(vecs, w):
        bank = w % 2
        copies = []
        for l4 in range(4):
            l = w * 4 + l4
            for t in range(6):
                i = vecs[cols[t]][l]
                off = pl.multiple_of((i >> 7) * 128, 128)
                copies.append(pltpu.async_copy(
                    tabs[t].at[:, pl.ds(off, 128)],
                    slabs[(bank * 4 + l4) * 6 + t],
                    sems[bank]))
        return copies

    def compute_wave(vecs, w):
        bank = w % 2
        for l4 in range(4):
            l = w * 4 + l4
            s = slabs[(bank * 4 + l4) * 6:]
            sub = [jnp.full((16,), vecs[c][l] & 127, jnp.int32)
                   for c in range(5)]
            rk = plsc.load_gather(s[0], [iota, sub[0]])
            lk = plsc.load_gather(s[1], [iota, sub[0]])
            pi = plsc.load_gather(s[2], [iota, sub[1]])
            pj = plsc.load_gather(s[3], [iota, sub[2]])
            ni = plsc.load_gather(s[4], [iota, sub[3]])
            nj = plsc.load_gather(s[5], [iota, sub[4]])
            t_ = (nj - ni - pj + pi) * rk + (ni * nj - pi * pj) * lk
            plsc.store_scatter(tbuf, [iota * 16 + l], t_)

    def group(g, carry):
        base = g * 16
        vecs = [
            plsc.load_gather(
                idx_v,
                [(c * BPW + base + iota) >> 7, (c * BPW + base + iota) & 127],
            )
            for c in range(5)
        ]
        prev = issue_wave(vecs, 0)
        for w in range(1, 4):
            nxt = issue_wave(vecs, w)
            for c in prev:
                c.wait()
            compute_wave(vecs, w - 1)
            prev = nxt
        for c in prev:
            c.wait()
        compute_wave(vecs, 3)
        acc = jnp.zeros((16,), jnp.float32)
        for d in range(16):
            acc = acc + plsc.load_gather(tbuf, [d * 16 + iota])
        flat = base + iota
        plsc.store_scatter(out_v, [flat >> 7, flat & 127], acc)
        return carry

    lax.fori_loop(0, NG, group, 0)
    pltpu.sync_copy(out_v, err_hbm.at[wid])


@functools.cache
def _sc_err():
    return pl.kernel(
        _sc_body,
        out_type=jax.ShapeDtypeStruct((NW, 4, 128), jnp.float32),
        mesh=_mesh(),
        scratch_types=[
            pltpu.VMEM((5 * BPW // 128, 128), jnp.int32),
            pltpu.VMEM((256,), jnp.float32),
            pltpu.VMEM((4, 128), jnp.float32),
            pltpu.SemaphoreType.DMA,
            pltpu.SemaphoreType.DMA,
        ] + [pltpu.VMEM((D, 128), jnp.float32)] * 48,
        compiler_params=pltpu.CompilerParams(needs_layout_passes=False),
    )


def _softplus_body(x_ref, o_ref):
    v = x_ref[...]
    o_ref[...] = jnp.maximum(v, 0.0) + jnp.log1p(jnp.exp(-jnp.abs(v)))


def _softplus_tc(err):
    x = err.reshape(128, 128)
    y = pl.pallas_call(
        _softplus_body,
        out_shape=jax.ShapeDtypeStruct((128, 128), jnp.float32),
    )(x)
    return y.reshape(B)


def kernel(batchVector, nodeEmbedding, relaEmbedding, linkEmbedding):
    idx = (batchVector.astype(jnp.int32)
           .reshape(NW, BPW, 5)
           .transpose(0, 2, 1)
           .reshape(NW, 5 * BPW // 128, 128))
    err = _sc_err()(idx, nodeEmbedding.T, relaEmbedding.T, linkEmbedding.T)
    return _softplus_tc(err.reshape(B))


# speedup vs baseline: 1.0664x; 1.0664x over previous
import functools

import jax
import jax.numpy as jnp
from jax import lax
from jax.experimental import pallas as pl
from jax.experimental.pallas import tpu as pltpu
from jax.experimental.pallas import tpu_sc as plsc

NC = 2
NS = 16
NW = NC * NS
B = 16384
D = 16
BPW = B // NW
NG = BPW // 16


@functools.cache
def _mesh():
    return plsc.VectorSubcoreMesh(
        core_axis_name="c", subcore_axis_name="s", num_cores=NC, num_subcores=NS
    )


def _sc_body(idx_hbm, nodeT, relaT, linkT, err_hbm, *scratch):
    idx_v = scratch[0]
    tbuf = scratch[1]
    out_v = scratch[2]
    sem = scratch[3]
    slabs = scratch[4:]
    wid = lax.axis_index("s") * NC + lax.axis_index("c")
    pltpu.sync_copy(idx_hbm.at[wid], idx_v)

    tabs = (relaT, linkT, nodeT, nodeT, nodeT, nodeT)
    cols = (0, 0, 1, 2, 3, 4)
    iota = lax.iota(jnp.int32, 16)

    def group(g, carry):
        base = g * 16
        vecs = [
            plsc.load_gather(
                idx_v,
                [(c * BPW + base + iota) >> 7, (c * BPW + base + iota) & 127],
            )
            for c in range(5)
        ]
        for half in range(2):
            copies = []
            for l8 in range(8):
                l = half * 8 + l8
                for t in range(6):
                    i = vecs[cols[t]][l]
                    off = pl.multiple_of((i >> 7) * 128, 128)
                    copies.append(pltpu.async_copy(
                        tabs[t].at[:, pl.ds(off, 128)], slabs[l8 * 6 + t], sem))
            for c in copies:
                c.wait()
            for l8 in range(8):
                l = half * 8 + l8
                sub = [jnp.full((16,), vecs[c][l] & 127, jnp.int32)
                       for c in range(5)]
                rk = plsc.load_gather(slabs[l8 * 6 + 0], [iota, sub[0]])
                lk = plsc.load_gather(slabs[l8 * 6 + 1], [iota, sub[0]])
                pi = plsc.load_gather(slabs[l8 * 6 + 2], [iota, sub[1]])
                pj = plsc.load_gather(slabs[l8 * 6 + 3], [iota, sub[2]])
                ni = plsc.load_gather(slabs[l8 * 6 + 4], [iota, sub[3]])
                nj = plsc.load_gather(slabs[l8 * 6 + 5], [iota, sub[4]])
                t_ = (nj - ni - pj + pi) * rk + (ni * nj - pi * pj) * lk
                plsc.store_scatter(tbuf, [iota * 16 + l], t_)
        acc = jnp.zeros((16,), jnp.float32)
        for d in range(16):
            acc = acc + plsc.load_gather(tbuf, [d * 16 + iota])
        flat = base + iota
        plsc.store_scatter(out_v, [flat >> 7, flat & 127], acc)
        return carry

    lax.fori_loop(0, NG, group, 0)
    pltpu.sync_copy(out_v, err_hbm.at[wid])


@functools.cache
def _sc_err():
    return pl.kernel(
        _sc_body,
        out_type=jax.ShapeDtypeStruct((NW, 4, 128), jnp.float32),
        mesh=_mesh(),
        scratch_types=[
            pltpu.VMEM((5 * BPW // 128, 128), jnp.int32),
            pltpu.VMEM((256,), jnp.float32),
            pltpu.VMEM((4, 128), jnp.float32),
            pltpu.SemaphoreType.DMA,
        ] + [pltpu.VMEM((D, 128), jnp.float32)] * 48,
        compiler_params=pltpu.CompilerParams(needs_layout_passes=False),
    )


def _softplus_body(x_ref, o_ref):
    v = x_ref[...]
    o_ref[...] = jnp.maximum(v, 0.0) + jnp.log1p(jnp.exp(-jnp.abs(v)))


def _softplus_tc(err):
    x = err.reshape(128, 128)
    y = pl.pallas_call(
        _softplus_body,
        out_shape=jax.ShapeDtypeStruct((128, 128), jnp.float32),
    )(x)
    return y.reshape(B)


def kernel(batchVector, nodeEmbedding, relaEmbedding, linkEmbedding):
    idx = (batchVector.astype(jnp.int32)
           .reshape(NW, BPW, 5)
           .transpose(0, 2, 1)
           .reshape(NW, 5 * BPW // 128, 128))
    err = _sc_err()(idx, nodeEmbedding.T, relaEmbedding.T, linkEmbedding.T)
    return _softplus_tc(err.reshape(B))
